# Initial kernel scaffold; baseline (speedup 1.0000x reference)
import jax
import jax.numpy as jnp
from jax.experimental import pallas as pl


def kernel(tokens, segments, token_table, segment_table, pos_w):
    raise NotImplementedError("write your pallas kernel here")


# speedup vs baseline: 5.0932x; 5.0932x over previous
import functools

import jax
import jax.numpy as jnp
from jax import lax
from jax.experimental import pallas as pl
from jax.experimental.pallas import tpu as pltpu
from jax.experimental.pallas import tpu_sc as plsc

VOCAB = 100000
HIDDEN = 128
MAXLEN = 200
BATCH = 1024

NTOK = BATCH * MAXLEN
NW = 32
TOK_PER_W = NTOK // NW
IDXROW = 128
ROWS_PER_W = TOK_PER_W // IDXROW
CHUNK_IR = 2
C = CHUNK_IR * IDXROW
NCHUNK = ROWS_PER_W // CHUNK_IR
NLANE = 16
VPC = C // NLANE


def _addtab_body(seg_ref, pos_ref, out_ref):
    out_ref[0:MAXLEN, :] = pos_ref[...] + seg_ref[0:1, :]
    out_ref[MAXLEN : 2 * MAXLEN, :] = pos_ref[...] + seg_ref[1:2, :]


def _build_addtab(segment_table, pos_w):
    return pl.pallas_call(
        _addtab_body,
        out_shape=jax.ShapeDtypeStruct((2 * MAXLEN, HIDDEN), jnp.float32),
    )(segment_table, pos_w)


_mesh = plsc.VectorSubcoreMesh(core_axis_name="c", subcore_axis_name="s")


@functools.partial(
    pl.kernel,
    mesh=_mesh,
    out_type=jax.ShapeDtypeStruct((NTOK, HIDDEN), jnp.float32),
    scratch_types=[
        pltpu.VMEM((CHUNK_IR, IDXROW), jnp.int32),
        pltpu.VMEM((CHUNK_IR, IDXROW), jnp.int32),
        pltpu.VMEM((C, HIDDEN), jnp.float32),
        pltpu.VMEM((C, HIDDEN), jnp.float32),
        pltpu.SemaphoreType.DMA,
    ],
)
def _emb(tok_tab, add_tab, tok_idx, seg_idx, out, tidx_v, aidx_v, trow_v, arow_v, sem):
    wid = lax.axis_index("s") * 2 + lax.axis_index("c")
    row0 = wid * ROWS_PER_W

    def chunk_body(cc, carry):
        r = row0 + cc * CHUNK_IR
        pltpu.sync_copy(tok_idx.at[pl.ds(r, CHUNK_IR)], tidx_v)
        pltpu.sync_copy(seg_idx.at[pl.ds(r, CHUNK_IR)], aidx_v)
        g0 = r * IDXROW
        for v in range(VPC):
            a, b = divmod(v * NLANE, IDXROW)
            sl = pl.ds(b, NLANE)
            g = g0 + v * NLANE + lax.iota(jnp.int32, NLANE)
            aidx_v[a, sl] = aidx_v[a, sl] * MAXLEN + lax.rem(g, MAXLEN)
        for k in range(CHUNK_IR):
            pltpu.async_copy(
                tok_tab.at[tidx_v.at[k]], trow_v.at[pl.ds(k * IDXROW, IDXROW)], sem
            ).wait()
            pltpu.async_copy(
                add_tab.at[aidx_v.at[k]], arow_v.at[pl.ds(k * IDXROW, IDXROW)], sem
            ).wait()

        def add_body(j, c2):
            for d in range(HIDDEN // NLANE):
                sl = pl.ds(d * NLANE, NLANE)
                trow_v[j, sl] = trow_v[j, sl] + arow_v[j, sl]
            return c2

        lax.fori_loop(0, C, add_body, 0)
        pltpu.sync_copy(trow_v, out.at[pl.ds(g0, C)])
        return carry

    lax.fori_loop(0, NCHUNK, chunk_body, 0)


def kernel(tokens, segments, token_table, segment_table, pos_w):
    tok_idx = tokens.astype(jnp.int32).reshape(NTOK // IDXROW, IDXROW)
    seg_idx = segments.astype(jnp.int32).reshape(NTOK // IDXROW, IDXROW)
    addtab = _build_addtab(segment_table, pos_w)
    out = _emb(token_table, addtab, tok_idx, seg_idx)
    return out.reshape(BATCH, MAXLEN, HIDDEN)


# speedup vs baseline: 5.9069x; 1.1598x over previous
import functools

import jax
import jax.numpy as jnp
from jax import lax
from jax.experimental import pallas as pl
from jax.experimental.pallas import tpu as pltpu
from jax.experimental.pallas import tpu_sc as plsc

VOCAB = 100000
HIDDEN = 128
MAXLEN = 200
BATCH = 1024

NTOK = BATCH * MAXLEN
NW = 32
TOK_PER_W = NTOK // NW
IDXROW = 128
ROWS_PER_W = TOK_PER_W // IDXROW
CHUNK_IR = 2
C = CHUNK_IR * IDXROW
NCHUNK = ROWS_PER_W // CHUNK_IR
NLANE = 16
VPC = C // NLANE


def _addtab_body(seg_ref, pos_ref, out_ref):
    out_ref[0:MAXLEN, :] = pos_ref[...] + seg_ref[0:1, :]
    out_ref[MAXLEN : 2 * MAXLEN, :] = pos_ref[...] + seg_ref[1:2, :]


def _build_addtab(segment_table, pos_w):
    return pl.pallas_call(
        _addtab_body,
        out_shape=jax.ShapeDtypeStruct((2 * MAXLEN, HIDDEN), jnp.float32),
    )(segment_table, pos_w)


_mesh = plsc.VectorSubcoreMesh(core_axis_name="c", subcore_axis_name="s")


@functools.partial(
    pl.kernel,
    mesh=_mesh,
    out_type=jax.ShapeDtypeStruct((NTOK, HIDDEN), jnp.float32),
    scratch_types=[
        pltpu.VMEM((CHUNK_IR, IDXROW), jnp.int32),
        pltpu.VMEM((CHUNK_IR, IDXROW), jnp.int32),
        pltpu.VMEM((C, HIDDEN), jnp.float32),
        pltpu.VMEM((C, HIDDEN), jnp.float32),
        pltpu.SemaphoreType.DMA,
    ],
)
def _emb(tok_tab, add_tab, tok_idx, seg_idx, out, tidx_v, aidx_v, trow_v, arow_v, sem):
    wid = lax.axis_index("s") * 2 + lax.axis_index("c")
    row0 = wid * ROWS_PER_W

    def chunk_body(cc, carry):
        r = row0 + cc * CHUNK_IR
        pltpu.sync_copy(tok_idx.at[pl.ds(r, CHUNK_IR)], tidx_v)
        pltpu.sync_copy(seg_idx.at[pl.ds(r, CHUNK_IR)], aidx_v)
        g0 = r * IDXROW
        for v in range(VPC):
            a, b = divmod(v * NLANE, IDXROW)
            sl = pl.ds(b, NLANE)
            g = g0 + v * NLANE + lax.iota(jnp.int32, NLANE)
            aidx_v[a, sl] = aidx_v[a, sl] * MAXLEN + lax.rem(g, MAXLEN)
        for k in range(CHUNK_IR):
            pltpu.async_copy(
                tok_tab.at[tidx_v.at[k]], trow_v.at[pl.ds(k * IDXROW, IDXROW)], sem
            ).wait()
            pltpu.sync_copy(
                add_tab.at[aidx_v.at[k]], trow_v.at[pl.ds(k * IDXROW, IDXROW)], add=True
            )
        pltpu.sync_copy(trow_v, out.at[pl.ds(g0, C)])
        return carry

    lax.fori_loop(0, NCHUNK, chunk_body, 0)


def kernel(tokens, segments, token_table, segment_table, pos_w):
    tok_idx = tokens.astype(jnp.int32).reshape(NTOK // IDXROW, IDXROW)
    seg_idx = segments.astype(jnp.int32).reshape(NTOK // IDXROW, IDXROW)
    addtab = _build_addtab(segment_table, pos_w)
    out = _emb(token_table, addtab, tok_idx, seg_idx)
    return out.reshape(BATCH, MAXLEN, HIDDEN)


# speedup vs baseline: 7.3570x; 1.2455x over previous
import functools

import jax
import jax.numpy as jnp
from jax import lax
from jax.experimental import pallas as pl
from jax.experimental.pallas import tpu as pltpu
from jax.experimental.pallas import tpu_sc as plsc

VOCAB = 100000
HIDDEN = 128
MAXLEN = 200
BATCH = 1024

NTOK = BATCH * MAXLEN
NW = 32
TOK_PER_W = NTOK // NW
C = 128
NCHUNK = TOK_PER_W // C
NLANE = 16
NBUF = 5
NSTEP = NCHUNK + 2
NITER = -(-NSTEP // NBUF)


def _addtab_body(seg_ref, pos_ref, out_ref):
    out_ref[0:MAXLEN, :] = pos_ref[...] + seg_ref[0:1, :]
    out_ref[MAXLEN : 2 * MAXLEN, :] = pos_ref[...] + seg_ref[1:2, :]


def _build_addtab(segment_table, pos_w):
    return pl.pallas_call(
        _addtab_body,
        out_shape=jax.ShapeDtypeStruct((2 * MAXLEN, HIDDEN), jnp.float32),
    )(segment_table, pos_w)


_mesh = plsc.VectorSubcoreMesh(core_axis_name="c", subcore_axis_name="s")


@functools.partial(
    pl.kernel,
    mesh=_mesh,
    out_type=jax.ShapeDtypeStruct((NTOK, HIDDEN), jnp.float32),
    scratch_types=[
        pltpu.VMEM((TOK_PER_W,), jnp.int32),
        pltpu.VMEM((TOK_PER_W,), jnp.int32),
    ]
    + [pltpu.VMEM((C, HIDDEN), jnp.float32) for _ in range(NBUF)]
    + [pltpu.SemaphoreType.DMA for _ in range(3 * NBUF)],
)
def _emb(tok_tab, add_tab, tok_idx, seg_idx, out, tidx_v, aidx_v, *scr):
    rows = scr[:NBUF]
    sem_g = scr[NBUF : 2 * NBUF]
    sem_a = scr[2 * NBUF : 3 * NBUF]
    sem_s = scr[3 * NBUF :]
    wid = lax.axis_index("s") * 2 + lax.axis_index("c")
    row0 = wid * NCHUNK
    tok0 = row0 * C

    pltpu.sync_copy(tok_idx.at[pl.ds(tok0, TOK_PER_W)], tidx_v)
    pltpu.sync_copy(seg_idx.at[pl.ds(tok0, TOK_PER_W)], aidx_v)

    def idx_body(j, carry):
        sl = pl.ds(j * NLANE, NLANE)
        g = tok0 + j * NLANE + lax.iota(jnp.int32, NLANE)
        aidx_v[sl] = aidx_v[sl] * MAXLEN + lax.rem(g, MAXLEN)
        return carry

    lax.fori_loop(0, TOK_PER_W // NLANE, idx_body, 0)

    def step_body(it, carry):
        for b in range(NBUF):
            s = it * NBUF + b
            bg, ba, bs = b, (b - 1) % NBUF, (b - 2) % NBUF

            @pl.when(jnp.logical_and(s >= NBUF, s < NCHUNK))
            def _():
                pltpu.make_async_copy(
                    rows[bg], out.at[pl.ds(0, C)], sem_s[bg]
                ).wait()

            @pl.when(s < NCHUNK)
            def _():
                pltpu.async_copy(
                    tok_tab.at[tidx_v.at[pl.ds(s * C, C)]], rows[bg], sem_g[bg]
                )

            @pl.when(jnp.logical_and(s >= 1, s <= NCHUNK))
            def _():
                c1 = s - 1
                pltpu.make_async_copy(
                    tok_tab.at[tidx_v.at[pl.ds(0, C)]], rows[ba], sem_g[ba]
                ).wait()
                pltpu.async_copy(
                    add_tab.at[aidx_v.at[pl.ds(c1 * C, C)]], rows[ba], sem_a[ba], add=True
                )

            @pl.when(jnp.logical_and(s >= 2, s <= NCHUNK + 1))
            def _():
                c2 = s - 2
                pltpu.make_async_copy(
                    add_tab.at[aidx_v.at[pl.ds(0, C)]], rows[bs], sem_a[bs]
                ).wait()
                pltpu.async_copy(rows[bs], out.at[pl.ds(tok0 + c2 * C, C)], sem_s[bs])

        return carry

    lax.fori_loop(0, NITER, step_body, 0)

    for c in range(NCHUNK - NBUF, NCHUNK):
        pltpu.make_async_copy(
            rows[c % NBUF], out.at[pl.ds(0, C)], sem_s[c % NBUF]
        ).wait()


def kernel(tokens, segments, token_table, segment_table, pos_w):
    tok_idx = tokens.astype(jnp.int32).reshape(NTOK)
    seg_idx = segments.astype(jnp.int32).reshape(NTOK)
    addtab = _build_addtab(segment_table, pos_w)
    out = _emb(token_table, addtab, tok_idx, seg_idx)
    return out.reshape(BATCH, MAXLEN, HIDDEN)


# speedup vs baseline: 13.9333x; 1.8939x over previous
import functools

import jax
import jax.numpy as jnp
from jax import lax
from jax.experimental import pallas as pl
from jax.experimental.pallas import tpu as pltpu
from jax.experimental.pallas import tpu_sc as plsc

VOCAB = 100000
HIDDEN = 128
MAXLEN = 200
BATCH = 1024

NTOK = BATCH * MAXLEN
NW = 32
TOK_PER_W = NTOK // NW
C = 128
NCHUNK = TOK_PER_W // C
NLANE = 16
NBUF = 5
NSTEP = NCHUNK + 2
NITER = -(-NSTEP // NBUF)


def _addtab_body(seg_ref, pos_ref, out_ref):
    out_ref[0:MAXLEN, :] = pos_ref[...] + seg_ref[0:1, :]
    out_ref[MAXLEN : 2 * MAXLEN, :] = pos_ref[...] + seg_ref[1:2, :]


def _build_addtab(segment_table, pos_w):
    return pl.pallas_call(
        _addtab_body,
        out_shape=jax.ShapeDtypeStruct((2 * MAXLEN, HIDDEN), jnp.float32),
    )(segment_table, pos_w)


_mesh = plsc.VectorSubcoreMesh(core_axis_name="c", subcore_axis_name="s")


@functools.partial(
    pl.kernel,
    mesh=_mesh,
    out_type=jax.ShapeDtypeStruct((NTOK, HIDDEN), jnp.float32),
    scratch_types=[
        pltpu.VMEM((TOK_PER_W,), jnp.int32),
        pltpu.VMEM((TOK_PER_W,), jnp.int32),
    ]
    + [pltpu.VMEM((C, HIDDEN), jnp.float32) for _ in range(NBUF)]
    + [pltpu.VMEM_SHARED((2 * MAXLEN, HIDDEN), jnp.float32)]
    + [pltpu.SemaphoreType.DMA for _ in range(3 * NBUF)],
)
def _emb(tok_tab, add_tab, tok_idx, seg_idx, out, tidx_v, aidx_v, *scr):
    rows = scr[:NBUF]
    addtab_sh = scr[NBUF]
    sem_g = scr[NBUF + 1 : 2 * NBUF + 1]
    sem_a = scr[2 * NBUF + 1 : 3 * NBUF + 1]
    sem_s = scr[3 * NBUF + 1 :]
    wid = lax.axis_index("s") * 2 + lax.axis_index("c")
    row0 = wid * NCHUNK
    tok0 = row0 * C

    pltpu.sync_copy(tok_idx.at[pl.ds(tok0, TOK_PER_W)], tidx_v)
    pltpu.sync_copy(seg_idx.at[pl.ds(tok0, TOK_PER_W)], aidx_v)

    def idx_body(j, carry):
        sl = pl.ds(j * NLANE, NLANE)
        g = tok0 + j * NLANE + lax.iota(jnp.int32, NLANE)
        aidx_v[sl] = aidx_v[sl] * MAXLEN + lax.rem(g, MAXLEN)
        return carry

    lax.fori_loop(0, TOK_PER_W // NLANE, idx_body, 0)

    @pl.when(lax.axis_index("s") == 0)
    def _():
        pltpu.sync_copy(add_tab, addtab_sh)

    plsc.subcore_barrier()

    def step_body(it, carry):
        for b in range(NBUF):
            s = it * NBUF + b
            bg, ba, bs = b, (b - 1) % NBUF, (b - 2) % NBUF

            @pl.when(jnp.logical_and(s >= NBUF, s < NCHUNK))
            def _():
                pltpu.make_async_copy(
                    rows[bg], out.at[pl.ds(0, C)], sem_s[bg]
                ).wait()

            @pl.when(s < NCHUNK)
            def _():
                pltpu.async_copy(
                    tok_tab.at[tidx_v.at[pl.ds(s * C, C)]], rows[bg], sem_g[bg]
                )

            @pl.when(jnp.logical_and(s >= 1, s <= NCHUNK))
            def _():
                c1 = s - 1
                pltpu.make_async_copy(
                    tok_tab.at[tidx_v.at[pl.ds(0, C)]], rows[ba], sem_g[ba]
                ).wait()
                pltpu.async_copy(
                    addtab_sh.at[aidx_v.at[pl.ds(c1 * C, C)]],
                    rows[ba],
                    sem_a[ba],
                    add=True,
                )

            @pl.when(jnp.logical_and(s >= 2, s <= NCHUNK + 1))
            def _():
                c2 = s - 2
                pltpu.make_async_copy(
                    addtab_sh.at[aidx_v.at[pl.ds(0, C)]], rows[bs], sem_a[bs]
                ).wait()
                pltpu.async_copy(rows[bs], out.at[pl.ds(tok0 + c2 * C, C)], sem_s[bs])

        return carry

    lax.fori_loop(0, NITER, step_body, 0)

    for c in range(NCHUNK - NBUF, NCHUNK):
        pltpu.make_async_copy(
            rows[c % NBUF], out.at[pl.ds(0, C)], sem_s[c % NBUF]
        ).wait()


def kernel(tokens, segments, token_table, segment_table, pos_w):
    tok_idx = tokens.astype(jnp.int32).reshape(NTOK)
    seg_idx = segments.astype(jnp.int32).reshape(NTOK)
    addtab = _build_addtab(segment_table, pos_w)
    out = _emb(token_table, addtab, tok_idx, seg_idx)
    return out.reshape(BATCH, MAXLEN, HIDDEN)
